# baseline (speedup 1.0000x reference)
import functools

import jax
import jax.numpy as jnp
from jax import lax
from jax.experimental import pallas as pl
from jax.experimental.pallas import tpu as pltpu
from jax.experimental.pallas import tpu_sc as plsc

BATCH = 64
BEAM = 5
T = 180
V = 1000
K = 10
LANES = 16
GROUPS = 20
CLAMP = V - LANES
TCHUNK = 8
NCORES = 2
NSUB = 16
NW = NCORES * NSUB
BPW = BATCH // NW

TPAD = 184
NEG = float(jnp.finfo(jnp.float32).min)
BIG = 2**30


def _iota16():
    return lax.iota(jnp.int32, LANES)


def _shuffle(xv, idx):
    dnums = lax.GatherDimensionNumbers(
        offset_dims=(), collapsed_slice_dims=(0,), start_index_map=(0,))
    return lax.gather(xv, idx[:, None], dnums, (1,),
                      mode=lax.GatherScatterMode.PROMISE_IN_BOUNDS)


def _merge_desc(a, b):
    ak, av = a
    bk = lax.rev(b[0], (0,))
    bv = lax.rev(b[1], (0,))
    take = ak >= bk
    mk = jnp.maximum(ak, bk)
    mv = jnp.where(take, av, bv)
    return plsc.sort_key_val(mk, mv, descending=True)


def _merge_desc_j(a, b):
    ak, av = a
    bk = lax.rev(b[0], (0,))
    bv = lax.rev(b[1], (0,))
    take = (ak > bk) | ((ak == bk) & (av < bv))
    mk = jnp.where(take, ak, bk)
    mv = jnp.where(take, av, bv)
    return plsc.sort_key_val(mk, mv, descending=True)


def _merge_tree(pairs, merge):
    while len(pairs) > 1:
        nxt = []
        for i in range(0, len(pairs) - 1, 2):
            nxt.append(merge(pairs[i], pairs[i + 1]))
        if len(pairs) % 2:
            nxt.append(pairs[-1])
        pairs = nxt
    return pairs[0]


def _sc_body(as_hbm, bs_hbm, out_v, out_b, out_w,
             buf, bsv0, bsv1, ovals, obeam, oword,
             sem_a, sem_b, sem_oa, sem_ob):
    wid = lax.axis_index("s") * NCORES + lax.axis_index("c")
    lane = _iota16()
    neg16 = jnp.full((LANES,), NEG, jnp.float32)

    def row_one(tt, pb, bi, par):
        partials = []
        for kb in range(BEAM):
            leaves = []
            for q in range(4):
                g = kb * 4 + q
                m0 = buf[pb + kb, tt, pl.ds(q * 256, LANES)]
                m1 = buf[pb + kb, tt, pl.ds(q * 256 + LANES, LANES)]
                for i in range(2, LANES, 2):
                    m0 = jnp.maximum(
                        m0, buf[pb + kb, tt, pl.ds(min(q * 256 + i * LANES,
                                                       CLAMP), LANES)])
                    m1 = jnp.maximum(
                        m1, buf[pb + kb, tt,
                                pl.ds(min(q * 256 + (i + 1) * LANES,
                                          CLAMP), LANES)])
                bsk = jnp.where(bi == 0, bsall[0][kb], bsall[1][kb])
                leaves.append(plsc.sort_key_val(
                    jnp.maximum(m0, m1) + bsk, lane + g * LANES,
                    descending=True))
            partials.append(_merge_tree(leaves, _merge_desc))
        _, colids = _merge_tree(partials, _merge_desc)

        g_ = lax.shift_right_logical(colids, 4)
        ln = colids & 15
        kv = lax.shift_right_logical(g_, 2)
        qv = g_ & 3
        base = qv * 256
        ttv = jnp.zeros((LANES,), jnp.int32) + tt
        bsg = jnp.where(bi == 0, bsall[0][0], bsall[1][0])
        for k in range(1, BEAM):
            bsg = jnp.where(kv == k,
                            jnp.where(bi == 0, bsall[0][k], bsall[1][k]),
                            bsg)
        cpairs = []
        for i in range(LANES):
            off = jnp.minimum(base + i * LANES, CLAMP)
            pos = off + ln
            cval = plsc.load_gather(buf, [kv + pb, ttv, pos]) + bsg
            if i == 14:
                cval = jnp.where((qv == 3) & (ln < 8), NEG, cval)
            elif i == 15:
                cval = jnp.where(qv == 3, NEG, cval)
            cpairs.append(plsc.sort_key_val(cval, kv * V + pos,
                                            descending=True))

        x, jv = _merge_tree(cpairs, _merge_desc_j)
        up = (lane + 1) & 15
        dn = (lane - 1) & 15
        for ph in range(4):
            x_up = _shuffle(x, up)
            j_up = _shuffle(jv, up)
            x_dn = _shuffle(x, dn)
            j_dn = _shuffle(jv, dn)
            lo = (lane & 1) == (ph & 1)
            take_up = lo & (x == x_up) & (jv > j_up)
            take_dn = (~lo) & (x == x_dn) & (j_dn > jv)
            jv = jnp.where(take_up, j_up, jnp.where(take_dn, j_dn, jv))
        beam = jv // V
        ovals[par, tt] = x
        obeam[par, tt] = beam
        oword[par, tt] = jv - beam * V

    NCHUNK = TPAD // TCHUNK
    TOTAL = BPW * NCHUNK

    def in_copies(w, par):
        bi = w // NCHUNK
        c = w - bi * NCHUNK
        batch = wid * BPW + bi
        t0 = pl.multiple_of(jnp.minimum(c * TCHUNK, TPAD - TCHUNK), TCHUNK)
        return [(as_hbm.at[batch * BEAM + k, pl.ds(t0, TCHUNK), :],
                 buf.at[par * BEAM + k]) for k in range(BEAM)]

    def out_copies(w, par):
        bi = w // NCHUNK
        c = w - bi * NCHUNK
        batch = wid * BPW + bi
        t0 = pl.multiple_of(jnp.minimum(c * TCHUNK, TPAD - TCHUNK), TCHUNK)
        dst = pl.ds(t0, TCHUNK)
        return [(ovals.at[par], out_v.at[batch, dst, :]),
                (obeam.at[par], out_b.at[batch, dst, :]),
                (oword.at[par], out_w.at[batch, dst, :])]

    pltpu.sync_copy(bs_hbm.at[wid * BPW], bsv0)
    pltpu.sync_copy(bs_hbm.at[wid * BPW + 1], bsv1)
    bsall = [[bsv0[k] for k in range(BEAM)],
             [bsv1[k] for k in range(BEAM)]]
    for s, d in in_copies(0, 0):
        pltpu.async_copy(s, d, sem_a)
    for s, d in in_copies(1, 1):
        pltpu.async_copy(s, d, sem_b)

    sems_in = (sem_a, sem_b)
    sems_out = (sem_oa, sem_ob)

    def half(u, par):
        w = u * 2 + par
        for s, d in in_copies(w, par):
            pltpu.make_async_copy(s, d, sems_in[par]).wait()

        @pl.when(u > 0)
        def _():
            for s, d in out_copies(w, par):
                pltpu.make_async_copy(s, d, sems_out[par]).wait()

        bi = w // NCHUNK

        def row_body(tt, c2):
            row_one(tt, par * BEAM, bi, par)
            return c2

        lax.fori_loop(0, TCHUNK, row_body, 0)
        for s, d in out_copies(w, par):
            pltpu.async_copy(s, d, sems_out[par])
        nxt = jnp.minimum(w + 2, TOTAL - 2 + par)
        for s, d in in_copies(nxt, par):
            pltpu.async_copy(s, d, sems_in[par])

    def work_body(u, carry):
        half(u, 0)
        half(u, 1)
        return carry

    lax.fori_loop(0, TOTAL // 2, work_body, 0)
    for s, d in in_copies(TOTAL - 2, 0):
        pltpu.make_async_copy(s, d, sem_a).wait()
    for s, d in in_copies(TOTAL - 1, 1):
        pltpu.make_async_copy(s, d, sem_b).wait()
    for s, d in out_copies(TOTAL - 2, 0):
        pltpu.make_async_copy(s, d, sem_oa).wait()
    for s, d in out_copies(TOTAL - 1, 1):
        pltpu.make_async_copy(s, d, sem_ob).wait()


@jax.jit
def kernel(all_scores, beam_scores):
    bs_pad = jnp.broadcast_to(beam_scores[:, :, None],
                              (BATCH, BEAM, LANES)).astype(jnp.float32)
    mesh = plsc.VectorSubcoreMesh(core_axis_name="c", subcore_axis_name="s",
                                  num_cores=NCORES, num_subcores=NSUB)
    out_type = (
        jax.ShapeDtypeStruct((BATCH, TPAD, LANES), jnp.float32),
        jax.ShapeDtypeStruct((BATCH, TPAD, LANES), jnp.int32),
        jax.ShapeDtypeStruct((BATCH, TPAD, LANES), jnp.int32),
    )
    run = pl.kernel(
        _sc_body,
        out_type,
        mesh=mesh,
        compiler_params=pltpu.CompilerParams(needs_layout_passes=False),
        scratch_types=[
            pltpu.VMEM((2 * BEAM, TCHUNK, V), jnp.float32),
            pltpu.VMEM((BEAM, LANES), jnp.float32),
            pltpu.VMEM((BEAM, LANES), jnp.float32),
            pltpu.VMEM((2, TCHUNK, LANES), jnp.float32),
            pltpu.VMEM((2, TCHUNK, LANES), jnp.int32),
            pltpu.VMEM((2, TCHUNK, LANES), jnp.int32),
            pltpu.SemaphoreType.DMA,
            pltpu.SemaphoreType.DMA,
            pltpu.SemaphoreType.DMA,
            pltpu.SemaphoreType.DMA,
        ],
    )
    vals, beams, words = run(all_scores, bs_pad)
    return (vals[:, 1:T, :K], beams[:, 1:T, :K], words[:, 1:T, :K])
